# baseline (speedup 1.0000x reference)
import math

import jax
import jax.numpy as jnp
from jax.experimental import pallas as pl
from jax.experimental.pallas import tpu as pltpu
from jax.experimental.pallas import tpu_sc as plsc

_W = 128
_LANES = 16


def kernel(x, table):
    B, S = x.shape
    N = B * S
    V, D = table.shape
    scale = float(math.sqrt(D))
    idx = x.reshape(1, N)

    mesh = plsc.VectorSubcoreMesh(core_axis_name="core",
                                  subcore_axis_name="subcore")

    @pl.kernel(out_type=jax.ShapeDtypeStruct((N, D), jnp.float32), mesh=mesh,
               compiler_params=pltpu.CompilerParams(use_tc_tiling_on_sc=False))
    def emb_kernel(table_hbm, i_hbm, o_hbm):
        def body(i_vmem, o_vmem):
            pltpu.sync_copy(table_hbm.at[i_vmem.at[0]], o_vmem)

            @pl.loop(0, _W)
            def _(r):
                for c in range(0, D, _LANES):
                    slc = (pl.ds(r, 1), pl.ds(c, _LANES))
                    o_vmem.at[slc][...] = o_vmem.at[slc][...] * scale

        pltpu.emit_pipeline(
            body,
            grid=(N // _W,),
            in_specs=[pl.BlockSpec((1, _W), index_map=lambda i: (0, i))],
            out_specs=[pl.BlockSpec((_W, D), index_map=lambda i: (i, 0))],
            core_axis_name=("core", "subcore"),
            dimension_semantics=(pltpu.PARALLEL,),
        )(i_hbm, o_hbm)

    out = emb_kernel(table, idx)
    return out.reshape(B, S, D)
